# baseline (speedup 1.0000x reference)
import jax
import jax.numpy as jnp
from jax.experimental import pallas as pl
from jax.experimental.pallas import tpu as pltpu

_BM = 200
_NSLOT = 3


def _mm_kernel(a_hbm, b_hbm, o_hbm, a_buf, b_vmem, o_vmem,
               a_sem, b_sem, o_sem):
    m = a_hbm.shape[0]
    num_panels = m // _BM

    def a_copy(i, slot):
        return pltpu.make_async_copy(
            a_hbm.at[pl.ds(i * _BM, _BM), :], a_buf.at[slot], a_sem.at[slot])

    pltpu.make_async_copy(b_hbm, b_vmem, b_sem).start()
    for s in range(_NSLOT):
        a_copy(s, s).start()
    pltpu.make_async_copy(b_hbm, b_vmem, b_sem).wait()

    def body(i, _):
        slot = jax.lax.rem(i, _NSLOT)
        a_copy(i, slot).wait()
        o_vmem[pl.ds(i * _BM, _BM), :] = jnp.dot(
            a_buf[slot], b_vmem[...], preferred_element_type=jnp.float32)

        @pl.when(i + _NSLOT < num_panels)
        def _prefetch():
            a_copy(i + _NSLOT, slot).start()

        return 0

    jax.lax.fori_loop(0, num_panels, body, 0)
    pltpu.make_async_copy(o_vmem, o_hbm, o_sem).start()
    pltpu.make_async_copy(o_vmem, o_hbm, o_sem).wait()


def kernel(feature, D_n_A_D_n):
    n, d = feature.shape
    m = D_n_A_D_n.shape[0]
    return pl.pallas_call(
        _mm_kernel,
        in_specs=[
            pl.BlockSpec(memory_space=pl.ANY),
            pl.BlockSpec(memory_space=pl.ANY),
        ],
        out_specs=pl.BlockSpec(memory_space=pl.ANY),
        out_shape=jax.ShapeDtypeStruct((m, d), jnp.float32),
        scratch_shapes=[
            pltpu.VMEM((_NSLOT, _BM, n), jnp.float32),
            pltpu.VMEM((n, d), jnp.float32),
            pltpu.VMEM((m, d), jnp.float32),
            pltpu.SemaphoreType.DMA((_NSLOT,)),
            pltpu.SemaphoreType.DMA,
            pltpu.SemaphoreType.DMA,
        ],
    )(D_n_A_D_n, feature)


# speedup vs baseline: 1.0034x; 1.0034x over previous
import jax
import jax.numpy as jnp
from jax.experimental import pallas as pl
from jax.experimental.pallas import tpu as pltpu

_BM = 200
_NSLOT = 3


def _mm_kernel(a_hbm, b_hbm, o_hbm, a_buf, b_vmem, o_buf,
               a_sem, b_sem, o_sem):
    m = a_hbm.shape[0]
    num_panels = m // _BM

    def a_copy(i, slot):
        return pltpu.make_async_copy(
            a_hbm.at[pl.ds(i * _BM, _BM), :], a_buf.at[slot], a_sem.at[slot])

    def o_copy(i, slot):
        return pltpu.make_async_copy(
            o_buf.at[slot], o_hbm.at[pl.ds(i * _BM, _BM), :], o_sem.at[slot])

    pltpu.make_async_copy(b_hbm, b_vmem, b_sem).start()
    for s in range(_NSLOT):
        a_copy(s, s).start()
    pltpu.make_async_copy(b_hbm, b_vmem, b_sem).wait()

    def body(i, _):
        slot = jax.lax.rem(i, _NSLOT)
        a_copy(i, slot).wait()

        @pl.when(i >= _NSLOT)
        def _drain():
            o_copy(i - _NSLOT, slot).wait()

        o_buf[slot] = jnp.dot(a_buf[slot], b_vmem[...],
                              preferred_element_type=jnp.float32)
        o_copy(i, slot).start()

        @pl.when(i + _NSLOT < num_panels)
        def _prefetch():
            a_copy(i + _NSLOT, slot).start()

        return 0

    jax.lax.fori_loop(0, num_panels, body, 0)
    for s in range(_NSLOT):
        i = num_panels - _NSLOT + s
        o_copy(i, jax.lax.rem(i, _NSLOT)).wait()


def kernel(feature, D_n_A_D_n):
    n, d = feature.shape
    m = D_n_A_D_n.shape[0]
    return pl.pallas_call(
        _mm_kernel,
        in_specs=[
            pl.BlockSpec(memory_space=pl.ANY),
            pl.BlockSpec(memory_space=pl.ANY),
        ],
        out_specs=pl.BlockSpec(memory_space=pl.ANY),
        out_shape=jax.ShapeDtypeStruct((m, d), jnp.float32),
        scratch_shapes=[
            pltpu.VMEM((_NSLOT, _BM, n), jnp.float32),
            pltpu.VMEM((n, d), jnp.float32),
            pltpu.VMEM((_NSLOT, _BM, d), jnp.float32),
            pltpu.SemaphoreType.DMA((_NSLOT,)),
            pltpu.SemaphoreType.DMA,
            pltpu.SemaphoreType.DMA((_NSLOT,)),
        ],
    )(D_n_A_D_n, feature)
